# Initial kernel scaffold; baseline (speedup 1.0000x reference)
import jax
import jax.numpy as jnp
from jax.experimental import pallas as pl


def kernel(indices, weight):
    raise NotImplementedError("write your pallas kernel here")


# speedup vs baseline: 1.0944x; 1.0944x over previous
import functools

import jax
import jax.numpy as jnp
from jax import lax
from jax.experimental import pallas as pl
from jax.experimental.pallas import tpu as pltpu
from jax.experimental.pallas import tpu_sc as plsc

NUM_EMB = 1_000_000
DIM = 32
B_TOTAL = 16384 * 50
NUM_CORES = 2
NUM_SUBCORES = 16
NW = NUM_CORES * NUM_SUBCORES
B_PER_W = B_TOTAL // NW
CHUNK = 1024
N_CHUNKS = B_PER_W // CHUNK

_MESH = plsc.VectorSubcoreMesh(core_axis_name="c", subcore_axis_name="s")


@functools.partial(
    pl.kernel,
    mesh=_MESH,
    out_type=jax.ShapeDtypeStruct((B_TOTAL, DIM), jnp.float32),
    scratch_types=[
        pltpu.VMEM((CHUNK,), jnp.int32),
        pltpu.VMEM((CHUNK, DIM), jnp.float32),
        pltpu.SemaphoreType.DMA,
    ],
    compiler_params=pltpu.CompilerParams(use_tc_tiling_on_sc=False),
)
def _gather_kernel(idx_hbm, table_hbm, out_hbm, idx_v, rows_v, sem):
    wid = lax.axis_index("s") * NUM_CORES + lax.axis_index("c")
    base = wid * B_PER_W

    def body(i, _):
        off = base + i * CHUNK
        pltpu.sync_copy(idx_hbm.at[pl.ds(off, CHUNK)], idx_v)
        pltpu.async_copy(table_hbm.at[idx_v], rows_v, sem).wait()
        pltpu.sync_copy(rows_v, out_hbm.at[pl.ds(off, CHUNK)])
        return 0

    lax.fori_loop(0, N_CHUNKS, body, 0)


def kernel(indices, weight):
    flat = indices.reshape(-1).astype(jnp.int32)
    out = _gather_kernel(flat, weight)
    return out.reshape(indices.shape + (DIM,))


# speedup vs baseline: 1.1089x; 1.0133x over previous
import functools

import jax
import jax.numpy as jnp
from jax import lax
from jax.experimental import pallas as pl
from jax.experimental.pallas import tpu as pltpu
from jax.experimental.pallas import tpu_sc as plsc

NUM_EMB = 1_000_000
DIM = 32
B_TOTAL = 16384 * 50
NUM_CORES = 2
NUM_SUBCORES = 16
NW = NUM_CORES * NUM_SUBCORES
B_PER_W = B_TOTAL // NW
CHUNK = 1600
N_CHUNKS = B_PER_W // CHUNK
N_BUF = 2

_MESH = plsc.VectorSubcoreMesh(core_axis_name="c", subcore_axis_name="s")


@functools.partial(
    pl.kernel,
    mesh=_MESH,
    out_type=jax.ShapeDtypeStruct((B_TOTAL, DIM), jnp.float32),
    scratch_types=[
        pltpu.VMEM((N_BUF, CHUNK), jnp.int32),
        pltpu.VMEM((N_BUF, CHUNK, DIM), jnp.float32),
        pltpu.SemaphoreType.DMA,
        pltpu.SemaphoreType.DMA,
        pltpu.SemaphoreType.DMA,
        pltpu.SemaphoreType.DMA,
    ],
    compiler_params=pltpu.CompilerParams(use_tc_tiling_on_sc=False),
)
def _gather_kernel(idx_hbm, table_hbm, out_hbm, idx_v, rows_v,
                   sem_g0, sem_g1, sem_s0, sem_s1):
    wid = lax.axis_index("s") * NUM_CORES + lax.axis_index("c")
    base = wid * B_PER_W
    sem_g = (sem_g0, sem_g1)
    sem_s = (sem_s0, sem_s1)

    def issue_gather(i, b):
        off = base + i * CHUNK
        pltpu.sync_copy(idx_hbm.at[pl.ds(off, CHUNK)], idx_v.at[b])
        pltpu.async_copy(table_hbm.at[idx_v.at[b]], rows_v.at[b], sem_g[b])

    def wait_gather(b):
        pltpu.make_async_copy(table_hbm.at[pl.ds(0, CHUNK)], rows_v.at[b],
                              sem_g[b]).wait()

    def issue_store(i, b):
        off = base + i * CHUNK
        pltpu.async_copy(rows_v.at[b], out_hbm.at[pl.ds(off, CHUNK)], sem_s[b])

    def wait_store(b):
        pltpu.make_async_copy(rows_v.at[b], out_hbm.at[pl.ds(0, CHUNK)],
                              sem_s[b]).wait()

    for b in range(N_BUF):
        issue_gather(b, b)

    def outer(g, _):
        for b in range(N_BUF):
            i = g * N_BUF + b
            wait_gather(b)
            issue_store(i, b)
            nxt = i + N_BUF

            @pl.when(nxt < N_CHUNKS)
            def _():
                wait_store(b)
                issue_gather(nxt, b)

        return 0

    lax.fori_loop(0, N_CHUNKS // N_BUF, outer, 0)
    for b in range(N_BUF):
        wait_store(b)


def kernel(indices, weight):
    flat = indices.reshape(-1).astype(jnp.int32)
    out = _gather_kernel(flat, weight)
    return out.reshape(indices.shape + (DIM,))


# speedup vs baseline: 1.6048x; 1.4472x over previous
import functools

import jax
import jax.numpy as jnp
from jax import lax
from jax.experimental import pallas as pl
from jax.experimental.pallas import tpu as pltpu
from jax.experimental.pallas import tpu_sc as plsc

DIM = 32
B0 = 16384
B1 = 50
NUM_CORES = 2
NUM_SUBCORES = 16
NW = NUM_CORES * NUM_SUBCORES
C0 = B0 // NW

_MESH = plsc.VectorSubcoreMesh(core_axis_name="c", subcore_axis_name="s")


@functools.partial(
    pl.kernel,
    mesh=_MESH,
    out_type=jax.ShapeDtypeStruct((B1, 4, 128, 8, 128), jnp.float32),
    scratch_types=[
        pltpu.VMEM((2, C0), jnp.int32),
        pltpu.VMEM((2, C0, DIM), jnp.float32),
        pltpu.VMEM((2, 4, 4, 8, 128), jnp.float32),
        pltpu.SemaphoreType.DMA,
        pltpu.SemaphoreType.DMA,
        pltpu.SemaphoreType.DMA,
        pltpu.SemaphoreType.DMA,
    ],
    compiler_params=pltpu.CompilerParams(use_tc_tiling_on_sc=False,
                                         needs_layout_passes=False),
)
def _gather_kernel(idx_hbm, table_hbm, out_hbm, idx_v, rows_v, trans_v,
                   sem_g0, sem_g1, sem_s0, sem_s1):
    wid = lax.axis_index("s") * NUM_CORES + lax.axis_index("c")
    base0 = wid * C0
    t0 = wid * 4
    sem_g = (sem_g0, sem_g1)
    sem_s = (sem_s0, sem_s1)
    lane = jax.lax.iota(jnp.int32, 16)

    def issue_gather(i1, b):
        pltpu.sync_copy(idx_hbm.at[i1, pl.ds(base0, C0)], idx_v.at[b])
        pltpu.async_copy(table_hbm.at[idx_v.at[b]], rows_v.at[b], sem_g[b])

    def wait_gather(b):
        pltpu.make_async_copy(table_hbm.at[pl.ds(0, C0)], rows_v.at[b],
                              sem_g[b]).wait()

    def issue_store(i1, b):
        pltpu.async_copy(trans_v.at[b],
                         out_hbm.at[i1, :, pl.ds(t0, 4), :, :], sem_s[b])

    def wait_store(b):
        pltpu.make_async_copy(trans_v.at[b],
                              out_hbm.at[0, :, pl.ds(0, 4), :, :],
                              sem_s[b]).wait()

    def transpose(b):
        rows = rows_v.at[b]
        trans = trans_v.at[b]

        def tr_body(st, _):
            s = st // 4
            tq = st % 4
            d_base = 8 * s
            v_base = tq * 128
            for r in range(8):
                d_idx = jnp.full((16,), d_base + r, jnp.int32)
                for c0 in range(8):
                    v_idx = lane + (v_base + c0 * 16)
                    vec = plsc.load_gather(rows, [v_idx, d_idx])
                    trans[s, tq, r, pl.ds(c0 * 16, 16)] = vec
            return 0

        lax.fori_loop(0, 16, tr_body, 0)

    for b in range(2):
        issue_gather(b, b)

    def outer(g, _):
        for b in range(2):
            i1 = 2 * g + b
            wait_gather(b)

            @pl.when(i1 >= 2)
            def _():
                wait_store(b)

            transpose(b)
            issue_store(i1, b)

            @pl.when(i1 + 2 < B1)
            def _():
                issue_gather(i1 + 2, b)

        return 0

    lax.fori_loop(0, B1 // 2, outer, 0)
    for b in range(2):
        wait_store(b)


def kernel(indices, weight):
    idx_t = indices.T.astype(jnp.int32)
    out5 = _gather_kernel(idx_t, weight)
    return jnp.transpose(out5, (2, 4, 0, 1, 3)).reshape(B0, B1, DIM)


# speedup vs baseline: 2.0193x; 1.2583x over previous
import functools

import jax
import jax.numpy as jnp
from jax import lax
from jax.experimental import pallas as pl
from jax.experimental.pallas import tpu as pltpu
from jax.experimental.pallas import tpu_sc as plsc

DIM = 32
B0 = 16384
B1 = 50
NUM_CORES = 2
NUM_SUBCORES = 16
NW = NUM_CORES * NUM_SUBCORES
C0 = B0 // NW

_MESH = plsc.VectorSubcoreMesh(core_axis_name="c", subcore_axis_name="s")


@functools.partial(
    pl.kernel,
    mesh=_MESH,
    out_type=jax.ShapeDtypeStruct((B1, 4, 128, 8, 128), jnp.float32),
    scratch_types=[
        pltpu.VMEM((B1, C0), jnp.int32),
        pltpu.VMEM((2, C0, DIM), jnp.float32),
        pltpu.VMEM((2, 4, 4, 8, 128), jnp.float32),
        pltpu.SemaphoreType.DMA,
        pltpu.SemaphoreType.DMA,
        pltpu.SemaphoreType.DMA,
        pltpu.SemaphoreType.DMA,
    ],
    compiler_params=pltpu.CompilerParams(use_tc_tiling_on_sc=False,
                                         needs_layout_passes=False),
)
def _gather_kernel(idx_hbm, table_hbm, out_hbm, idx_v, rows_v, trans_v,
                   sem_g0, sem_g1, sem_s0, sem_s1):
    wid = lax.axis_index("s") * NUM_CORES + lax.axis_index("c")
    base0 = wid * C0
    t0 = wid * 4
    sem_g = (sem_g0, sem_g1)
    sem_s = (sem_s0, sem_s1)
    lane = lax.iota(jnp.int32, 16)

    pltpu.sync_copy(idx_hbm.at[:, pl.ds(base0, C0)], idx_v)

    def issue_gather(i1, b):
        pltpu.async_copy(table_hbm.at[idx_v.at[i1]], rows_v.at[b], sem_g[b])

    def wait_gather(b):
        pltpu.make_async_copy(table_hbm.at[pl.ds(0, C0)], rows_v.at[b],
                              sem_g[b]).wait()

    def issue_store(i1, b):
        pltpu.async_copy(trans_v.at[b],
                         out_hbm.at[i1, :, pl.ds(t0, 4), :, :], sem_s[b])

    def wait_store(b):
        pltpu.make_async_copy(trans_v.at[b],
                              out_hbm.at[0, :, pl.ds(0, 4), :, :],
                              sem_s[b]).wait()

    def transpose(b):
        rows = rows_v.at[b]
        trans = trans_v.at[b]

        @plsc.parallel_loop(0, 128, unroll=2)
        def _tr(it):
            tq = it & 3
            d = it >> 2
            s = d >> 3
            r = d & 7
            vb = tq * 128
            for c0 in range(8):
                v_idx = lane + (vb + c0 * 16)
                d_idx = jnp.full((16,), d, jnp.int32)
                vec = plsc.load_gather(rows, [v_idx, d_idx])
                trans[s, tq, r, pl.ds(c0 * 16, 16)] = vec

    for b in range(2):
        issue_gather(b, b)

    def outer(g, _):
        for b in range(2):
            i1 = 2 * g + b
            wait_gather(b)

            @pl.when(i1 >= 2)
            def _():
                wait_store(b)

            transpose(b)
            issue_store(i1, b)

            @pl.when(i1 + 2 < B1)
            def _():
                issue_gather(i1 + 2, b)

        return 0

    lax.fori_loop(0, B1 // 2, outer, 0)
    for b in range(2):
        wait_store(b)


def kernel(indices, weight):
    idx_t = indices.T.astype(jnp.int32)
    out5 = _gather_kernel(idx_t, weight)
    return jnp.transpose(out5, (2, 4, 0, 1, 3)).reshape(B0, B1, DIM)


# speedup vs baseline: 2.1302x; 1.0549x over previous
import functools

import jax
import jax.numpy as jnp
from jax import lax
from jax.experimental import pallas as pl
from jax.experimental.pallas import tpu as pltpu
from jax.experimental.pallas import tpu_sc as plsc

DIM = 32
B0 = 16384
B1 = 50
NROWS = 1_000_000
NUM_CORES = 2
NUM_SUBCORES = 16
NW = NUM_CORES * NUM_SUBCORES
C0 = B0 // NW

NBLK = NROWS // 128
BLK_EVEN = NBLK // NW
BLK_TAIL = NBLK - BLK_EVEN * NW
TAIL_ROWS = NROWS - NBLK * 128

_MESH = plsc.VectorSubcoreMesh(core_axis_name="c", subcore_axis_name="s")


@functools.partial(
    pl.kernel,
    mesh=_MESH,
    out_type=jax.ShapeDtypeStruct((NROWS * DIM,), jnp.float32),
    scratch_types=[
        pltpu.VMEM((4, 8, 128), jnp.float32),
        pltpu.VMEM((4, 8, 128), jnp.float32),
        pltpu.VMEM((4096,), jnp.float32),
        pltpu.VMEM((4096,), jnp.float32),
        pltpu.SemaphoreType.DMA,
        pltpu.SemaphoreType.DMA,
        pltpu.SemaphoreType.DMA,
        pltpu.SemaphoreType.DMA,
    ],
    compiler_params=pltpu.CompilerParams(needs_layout_passes=False),
)
def _convert_kernel(w_hbm, tail_hbm, out_hbm, in_v0, in_v1, t_v0, t_v1,
                    sem_i0, sem_i1, sem_o0, sem_o1):
    wid = lax.axis_index("s") * NUM_CORES + lax.axis_index("c")
    cb0 = wid * BLK_EVEN
    in_v = (in_v0, in_v1)
    t_v = (t_v0, t_v1)
    sem_i = (sem_i0, sem_i1)
    sem_o = (sem_o0, sem_o1)
    lane32 = lax.iota(jnp.int32, 16) * DIM

    def issue_in(j, b):
        cb = cb0 + j
        pltpu.async_copy(w_hbm.at[:, :, pl.ds(cb * 128, 128)], in_v[b],
                         sem_i[b])

    def wait_in(b):
        pltpu.make_async_copy(w_hbm.at[:, :, pl.ds(0, 128)], in_v[b],
                              sem_i[b]).wait()

    def issue_out(j, b):
        cb = cb0 + j
        pltpu.async_copy(t_v[b], out_hbm.at[pl.ds(cb * 4096, 4096)],
                         sem_o[b])

    def wait_out(b):
        pltpu.make_async_copy(t_v[b], out_hbm.at[pl.ds(0, 4096)],
                              sem_o[b]).wait()

    def transpose(b):
        src = in_v[b]
        dst = t_v[b]

        @plsc.parallel_loop(0, 32, unroll=2)
        def _tr(g):
            t = g >> 3
            d8 = g & 7
            base = 8 * t + d8
            for b0 in range(8):
                vec = src[t, d8, pl.ds(16 * b0, 16)]
                plsc.store_scatter(dst, [lane32 + (512 * b0 + base)], vec)

    for b in range(2):
        issue_in(b, b)

    def outer(g, _):
        for b in range(2):
            j = 2 * g + b
            wait_in(b)

            @pl.when(j >= 2)
            def _():
                wait_out(b)

            transpose(b)
            issue_out(j, b)

            @pl.when(j + 2 < BLK_EVEN)
            def _():
                issue_in(j + 2, b)

        return 0

    lax.fori_loop(0, BLK_EVEN // 2, outer, 0)
    for b in range(2):
        wait_out(b)

    @pl.when(wid < BLK_TAIL)
    def _():
        cb = NW * BLK_EVEN + wid
        pltpu.sync_copy(w_hbm.at[:, :, pl.ds(cb * 128, 128)], in_v0)
        transpose(0)
        pltpu.sync_copy(t_v0, out_hbm.at[pl.ds(cb * 4096, 4096)])

    @pl.when(wid == BLK_TAIL)
    def _():
        pltpu.sync_copy(
            tail_hbm,
            out_hbm.at[pl.ds(NBLK * 4096, TAIL_ROWS * DIM)])


@functools.partial(
    pl.kernel,
    mesh=_MESH,
    out_type=jax.ShapeDtypeStruct((B1, 4, 128, 8, 128), jnp.float32),
    scratch_types=[
        pltpu.VMEM((B1, C0), jnp.int32),
        pltpu.VMEM((2, C0, DIM), jnp.float32),
        pltpu.VMEM((2, 4, 4, 8, 128), jnp.float32),
        pltpu.SemaphoreType.DMA,
        pltpu.SemaphoreType.DMA,
        pltpu.SemaphoreType.DMA,
        pltpu.SemaphoreType.DMA,
    ],
    compiler_params=pltpu.CompilerParams(use_tc_tiling_on_sc=False,
                                         needs_layout_passes=False),
)
def _gather_kernel(idx_hbm, table_hbm, out_hbm, idx_v, rows_v, trans_v,
                   sem_g0, sem_g1, sem_s0, sem_s1):
    wid = lax.axis_index("s") * NUM_CORES + lax.axis_index("c")
    base0 = wid * C0
    t0 = wid * 4
    sem_g = (sem_g0, sem_g1)
    sem_s = (sem_s0, sem_s1)
    lane = lax.iota(jnp.int32, 16)

    pltpu.sync_copy(idx_hbm.at[:, pl.ds(base0, C0)], idx_v)

    def issue_gather(i1, b):
        pltpu.async_copy(table_hbm.at[idx_v.at[i1]], rows_v.at[b], sem_g[b])

    def wait_gather(b):
        pltpu.make_async_copy(table_hbm.at[pl.ds(0, C0)], rows_v.at[b],
                              sem_g[b]).wait()

    def issue_store(i1, b):
        pltpu.async_copy(trans_v.at[b],
                         out_hbm.at[i1, :, pl.ds(t0, 4), :, :], sem_s[b])

    def wait_store(b):
        pltpu.make_async_copy(trans_v.at[b],
                              out_hbm.at[0, :, pl.ds(0, 4), :, :],
                              sem_s[b]).wait()

    def transpose(b):
        rows = rows_v.at[b]
        trans = trans_v.at[b]

        @plsc.parallel_loop(0, 128, unroll=2)
        def _tr(it):
            tq = it & 3
            d = it >> 2
            s = d >> 3
            r = d & 7
            vb = tq * 128
            for c0 in range(8):
                v_idx = lane + (vb + c0 * 16)
                d_idx = jnp.full((16,), d, jnp.int32)
                vec = plsc.load_gather(rows, [v_idx, d_idx])
                trans[s, tq, r, pl.ds(c0 * 16, 16)] = vec

    for b in range(2):
        issue_gather(b, b)

    def outer(g, _):
        for b in range(2):
            i1 = 2 * g + b
            wait_gather(b)

            @pl.when(i1 >= 2)
            def _():
                wait_store(b)

            transpose(b)
            issue_store(i1, b)

            @pl.when(i1 + 2 < B1)
            def _():
                issue_gather(i1 + 2, b)

        return 0

    lax.fori_loop(0, B1 // 2, outer, 0)
    for b in range(2):
        wait_store(b)


def kernel(indices, weight):
    idx_t = indices.T.astype(jnp.int32)
    w_native = weight.T.reshape(4, 8, NROWS)
    tail = weight[NBLK * 128:].reshape(TAIL_ROWS * DIM)
    table = _convert_kernel(w_native, tail)
    table_rm = table.reshape(NROWS, DIM)
    out5 = _gather_kernel(idx_t, table_rm)
    return jnp.transpose(out5, (2, 4, 0, 1, 3)).reshape(B0, B1, DIM)
